# baseline (speedup 1.0000x reference)
import jax
import jax.numpy as jnp
from jax import lax
from jax.experimental import pallas as pl
from jax.experimental.pallas import tpu as pltpu
from jax.experimental.pallas import tpu_sc as plsc

_L = 16


def _sc_body(c_hbm, w_hbm, b_hbm, out_hbm, c_v, w_v, b_v, o_v):
    cid = lax.axis_index("c")
    sid = lax.axis_index("s")
    pltpu.sync_copy(c_hbm, c_v)
    pltpu.sync_copy(w_hbm, w_v)
    pltpu.sync_copy(b_hbm, b_v)
    gathered = plsc.load_gather(w_v, [c_v[...]])
    o_v[...] = gathered + b_v[...]

    @pl.when(jnp.logical_and(cid == 0, sid == 0))
    def _():
        pltpu.sync_copy(o_v, out_hbm)


def kernel(c, W, b):
    c_vec = jnp.full((_L,), c, dtype=jnp.int32)
    w_vec = jnp.pad(W.reshape(-1), (0, _L - W.size))
    b_vec = jnp.broadcast_to(b, (_L,))
    mesh = plsc.VectorSubcoreMesh(
        core_axis_name="c", subcore_axis_name="s", num_cores=2, num_subcores=16
    )
    out16 = pl.kernel(
        _sc_body,
        out_type=jax.ShapeDtypeStruct((_L,), jnp.float32),
        mesh=mesh,
        compiler_params=pltpu.CompilerParams(needs_layout_passes=False),
        scratch_types=[
            pltpu.VMEM((_L,), jnp.int32),
            pltpu.VMEM((_L,), jnp.float32),
            pltpu.VMEM((_L,), jnp.float32),
            pltpu.VMEM((_L,), jnp.float32),
        ],
    )(c_vec, w_vec, b_vec)
    return out16[:1]


# speedup vs baseline: 1.0444x; 1.0444x over previous
import jax
import jax.numpy as jnp
from jax import lax
from jax.experimental import pallas as pl
from jax.experimental.pallas import tpu as pltpu
from jax.experimental.pallas import tpu_sc as plsc

_L = 16


def _sc_body(c_hbm, wb_hbm, out_hbm, c_v, wb_v, o_v):
    cid = lax.axis_index("c")
    sid = lax.axis_index("s")
    pltpu.sync_copy(c_hbm, c_v)
    pltpu.sync_copy(wb_hbm, wb_v)
    w_c = plsc.load_gather(wb_v, [c_v[...]])
    b_s = plsc.load_gather(wb_v, [jnp.full((_L,), 10, jnp.int32)])
    o_v[...] = w_c + b_s

    @pl.when(jnp.logical_and(cid == 0, sid == 0))
    def _():
        pltpu.sync_copy(o_v, out_hbm)


def kernel(c, W, b):
    c_vec = jnp.full((_L,), c, dtype=jnp.int32)
    wb_vec = jnp.concatenate([W.reshape(-1), b, jnp.zeros((_L - 11,), jnp.float32)])
    mesh = plsc.VectorSubcoreMesh(
        core_axis_name="c", subcore_axis_name="s", num_cores=2, num_subcores=16
    )
    out16 = pl.kernel(
        _sc_body,
        out_type=jax.ShapeDtypeStruct((_L,), jnp.float32),
        mesh=mesh,
        compiler_params=pltpu.CompilerParams(needs_layout_passes=False),
        scratch_types=[
            pltpu.VMEM((_L,), jnp.int32),
            pltpu.VMEM((_L,), jnp.float32),
            pltpu.VMEM((_L,), jnp.float32),
        ],
    )(c_vec, wb_vec)
    return out16[:1]


# speedup vs baseline: 1.1126x; 1.0654x over previous
import jax
import jax.numpy as jnp
from jax import lax
from jax.experimental import pallas as pl
from jax.experimental.pallas import tpu as pltpu
from jax.experimental.pallas import tpu_sc as plsc

_L = 16


def _sc_body(c_hbm, wb_hbm, out_hbm, c_v, wb_v, o_v):
    cid = lax.axis_index("c")
    sid = lax.axis_index("s")

    @pl.when(jnp.logical_and(cid == 0, sid == 0))
    def _():
        pltpu.sync_copy(c_hbm, c_v)
        pltpu.sync_copy(wb_hbm, wb_v)
        w_c = plsc.load_gather(wb_v, [c_v[...]])
        b_s = plsc.load_gather(wb_v, [jnp.full((_L,), 10, jnp.int32)])
        o_v[...] = w_c + b_s
        pltpu.sync_copy(o_v, out_hbm)


def kernel(c, W, b):
    c_vec = jnp.full((_L,), c, dtype=jnp.int32)
    wb_vec = jnp.concatenate([W.reshape(-1), b, jnp.zeros((_L - 11,), jnp.float32)])
    mesh = plsc.VectorSubcoreMesh(
        core_axis_name="c", subcore_axis_name="s", num_cores=2, num_subcores=16
    )
    out16 = pl.kernel(
        _sc_body,
        out_type=jax.ShapeDtypeStruct((_L,), jnp.float32),
        mesh=mesh,
        compiler_params=pltpu.CompilerParams(needs_layout_passes=False),
        scratch_types=[
            pltpu.VMEM((_L,), jnp.int32),
            pltpu.VMEM((_L,), jnp.float32),
            pltpu.VMEM((_L,), jnp.float32),
        ],
    )(c_vec, wb_vec)
    return out16[:1]


# speedup vs baseline: 1.1954x; 1.0744x over previous
import jax
import jax.numpy as jnp
from jax import lax
from jax.experimental import pallas as pl
from jax.experimental.pallas import tpu as pltpu
from jax.experimental.pallas import tpu_sc as plsc

_L = 16


def _sc_body(c_hbm, wb_hbm, out_hbm, c_v, wb_v, o_v):
    cid = lax.axis_index("c")
    sid = lax.axis_index("s")

    @pl.when(jnp.logical_and(cid == 0, sid == 0))
    def _():
        pltpu.sync_copy(c_hbm, c_v)
        pltpu.sync_copy(wb_hbm, wb_v)
        w_c = plsc.load_gather(wb_v, [c_v[...]])
        b_s = plsc.load_gather(wb_v, [jnp.full((_L,), 10, jnp.int32)])
        o_v[...] = w_c + b_s
        pltpu.sync_copy(o_v, out_hbm)


def kernel(c, W, b):
    c_vec = jnp.full((_L,), c, dtype=jnp.int32)
    wb_vec = jnp.concatenate([W.reshape(-1), b, jnp.zeros((_L - 11,), jnp.float32)])
    mesh = plsc.VectorSubcoreMesh(
        core_axis_name="c", subcore_axis_name="s", num_cores=1, num_subcores=16
    )
    out16 = pl.kernel(
        _sc_body,
        out_type=jax.ShapeDtypeStruct((_L,), jnp.float32),
        mesh=mesh,
        compiler_params=pltpu.CompilerParams(needs_layout_passes=False),
        scratch_types=[
            pltpu.VMEM((_L,), jnp.int32),
            pltpu.VMEM((_L,), jnp.float32),
            pltpu.VMEM((_L,), jnp.float32),
        ],
    )(c_vec, wb_vec)
    return out16[:1]


# speedup vs baseline: 1.2045x; 1.0076x over previous
import jax
import jax.numpy as jnp
from jax import lax
from jax.experimental import pallas as pl
from jax.experimental.pallas import tpu as pltpu
from jax.experimental.pallas import tpu_sc as plsc

_L = 16


def _sc_body(c_hbm, wb_hbm, out_hbm, c_v, wb_v, o_v):
    cid = lax.axis_index("c")
    sid = lax.axis_index("s")

    @pl.when(jnp.logical_and(cid == 0, sid == 0))
    def _():
        pltpu.sync_copy(c_hbm, c_v)
        pltpu.sync_copy(wb_hbm, wb_v)
        w_c = plsc.load_gather(wb_v, [c_v[...]])
        b_s = plsc.load_gather(wb_v, [jnp.full((_L,), 10, jnp.int32)])
        o_v[...] = w_c + b_s
        pltpu.sync_copy(o_v, out_hbm)


def kernel(c, W, b):
    c_vec = jnp.full((_L,), c, dtype=jnp.int32)
    wb_vec = jnp.concatenate([W.reshape(-1), b, jnp.zeros((_L - 11,), jnp.float32)])
    mesh = plsc.VectorSubcoreMesh(
        core_axis_name="c", subcore_axis_name="s", num_cores=1, num_subcores=1
    )
    out16 = pl.kernel(
        _sc_body,
        out_type=jax.ShapeDtypeStruct((_L,), jnp.float32),
        mesh=mesh,
        compiler_params=pltpu.CompilerParams(needs_layout_passes=False),
        scratch_types=[
            pltpu.VMEM((_L,), jnp.int32),
            pltpu.VMEM((_L,), jnp.float32),
            pltpu.VMEM((_L,), jnp.float32),
        ],
    )(c_vec, wb_vec)
    return out16[:1]


# speedup vs baseline: 1.2097x; 1.0044x over previous
import jax
import jax.numpy as jnp
from jax import lax
from jax.experimental import pallas as pl
from jax.experimental.pallas import tpu as pltpu
from jax.experimental.pallas import tpu_sc as plsc

_L = 16


def _sc_body(c_hbm, wb_hbm, out_hbm, c_v, wb_v, o_v):
    cid = lax.axis_index("c")
    sid = lax.axis_index("s")

    @pl.when(jnp.logical_and(cid == 0, sid == 0))
    def _():
        pltpu.sync_copy(c_hbm, c_v)
        pltpu.sync_copy(wb_hbm, wb_v)
        w_c = plsc.load_gather(wb_v, [c_v[...]])
        b_s = plsc.load_gather(wb_v, [jnp.full((_L,), 10, jnp.int32)])
        o_v[...] = w_c + b_s
        pltpu.sync_copy(o_v, out_hbm)


def kernel(c, W, b):
    c_vec = jnp.full((_L,), c, dtype=jnp.int32)
    wb_vec = jnp.concatenate([W.reshape(-1), b, jnp.zeros((_L - 11,), jnp.float32)])
    mesh = plsc.VectorSubcoreMesh(
        core_axis_name="c", subcore_axis_name="s", num_cores=1, num_subcores=1
    )
    out16 = pl.kernel(
        _sc_body,
        out_type=jax.ShapeDtypeStruct((_L,), jnp.float32),
        mesh=mesh,
        compiler_params=pltpu.CompilerParams(
            needs_layout_passes=False, skip_device_barrier=True
        ),
        scratch_types=[
            pltpu.VMEM((_L,), jnp.int32),
            pltpu.VMEM((_L,), jnp.float32),
            pltpu.VMEM((_L,), jnp.float32),
        ],
    )(c_vec, wb_vec)
    return out16[:1]


# speedup vs baseline: 1.2682x; 1.0483x over previous
import jax
import jax.numpy as jnp
from jax import lax
from jax.experimental import pallas as pl
from jax.experimental.pallas import tpu as pltpu
from jax.experimental.pallas import tpu_sc as plsc

_L = 16


def _sc_body(p_hbm, out_hbm, p_v, o_v):
    cid = lax.axis_index("c")
    sid = lax.axis_index("s")

    @pl.when(jnp.logical_and(cid == 0, sid == 0))
    def _():
        pltpu.sync_copy(p_hbm, p_v)
        p = p_v[...]
        pi = plsc.bitcast(p, jnp.int32)
        idx_c = pi.at[jnp.full((_L,), 11, jnp.int32)].get(mode="promise_in_bounds")
        w_c = p.at[idx_c].get(mode="promise_in_bounds")
        b_s = p.at[jnp.full((_L,), 10, jnp.int32)].get(mode="promise_in_bounds")
        o_v[...] = w_c + b_s
        pltpu.sync_copy(o_v, out_hbm)


def kernel(c, W, b):
    c_f = lax.bitcast_convert_type(jnp.asarray(c, jnp.int32), jnp.float32).reshape(1)
    packed = jnp.concatenate(
        [W.reshape(-1), b, c_f, jnp.zeros((_L - 12,), jnp.float32)]
    )
    mesh = plsc.VectorSubcoreMesh(
        core_axis_name="c", subcore_axis_name="s", num_cores=1, num_subcores=1
    )
    out16 = pl.kernel(
        _sc_body,
        out_type=jax.ShapeDtypeStruct((_L,), jnp.float32),
        mesh=mesh,
        compiler_params=pltpu.CompilerParams(
            needs_layout_passes=False, skip_device_barrier=True
        ),
        scratch_types=[
            pltpu.VMEM((_L,), jnp.float32),
            pltpu.VMEM((_L,), jnp.float32),
        ],
    )(packed)
    return out16[:1]
